# baseline (speedup 1.0000x reference)
import functools
import math

import jax
import jax.numpy as jnp
from jax import lax
from jax.experimental import pallas as pl
from jax.experimental.pallas import tpu as pltpu
from jax.experimental.pallas import tpu_sc as plsc

D_MODEL = 64
SCALE = math.sqrt(D_MODEL)
LANES = 16
NUM_CORES = 2
NUM_SUBCORES = 16
NUM_WORKERS = NUM_CORES * NUM_SUBCORES


@functools.lru_cache(maxsize=None)
def _build(n_total: int, vocab: int, d: int, chunk: int):
    per_w = n_total // NUM_WORKERS
    n_chunks = per_w // chunk
    slices_per_row = d // LANES

    mesh = plsc.VectorSubcoreMesh(core_axis_name="c", subcore_axis_name="s")

    @functools.partial(
        pl.kernel,
        out_type=jax.ShapeDtypeStruct((n_total, d), jnp.float32),
        mesh=mesh,
        scratch_types=[
            pltpu.VMEM((per_w,), jnp.int32),
            pltpu.VMEM((chunk, d), jnp.float32),
            pltpu.SemaphoreType.DMA,
        ],
        compiler_params=pltpu.CompilerParams(use_tc_tiling_on_sc=False),
    )
    def emb_kernel(x_hbm, table_hbm, out_hbm, idx_v, rows_v, gsem):
        wid = lax.axis_index("s") * NUM_CORES + lax.axis_index("c")
        base = wid * per_w
        pltpu.sync_copy(x_hbm.at[pl.ds(base, per_w)], idx_v)

        def chunk_body(c, carry):
            cb = c * chunk
            pltpu.async_copy(
                table_hbm.at[idx_v.at[pl.ds(cb, chunk)]], rows_v, gsem
            ).wait()

            def scale_body(r, carry2):
                for j in range(slices_per_row):
                    sl = pl.ds(j * LANES, LANES)
                    rows_v[r, sl] = rows_v[r, sl] * SCALE
                return carry2

            lax.fori_loop(0, chunk, scale_body, 0)
            pltpu.sync_copy(rows_v, out_hbm.at[pl.ds(base + cb, chunk)])
            return carry

        lax.fori_loop(0, n_chunks, chunk_body, 0)

    return emb_kernel


def _pick_chunk(per_w: int) -> int:
    best = None
    for c in range(8, per_w + 1, 8):
        if per_w % c:
            continue
        if best is None or abs(c - 640) < abs(best - 640):
            best = c
    return best if best is not None else per_w


def kernel(x, table):
    orig_shape = x.shape
    xf = x.reshape(-1).astype(jnp.int32)
    n_total = xf.shape[0]
    vocab, d = table.shape
    assert n_total % NUM_WORKERS == 0
    per_w = n_total // NUM_WORKERS
    chunk = _pick_chunk(per_w)
    out = _build(n_total, vocab, d, chunk)(xf, table)
    return out.reshape(orig_shape + (d,))


# speedup vs baseline: 1.0413x; 1.0413x over previous
import functools
import math

import jax
import jax.numpy as jnp
from jax import lax
from jax.experimental import pallas as pl
from jax.experimental.pallas import tpu as pltpu
from jax.experimental.pallas import tpu_sc as plsc

D_MODEL = 64
SCALE = math.sqrt(D_MODEL)
LANES = 16
NUM_CORES = 2
NUM_SUBCORES = 16
NUM_WORKERS = NUM_CORES * NUM_SUBCORES


@functools.lru_cache(maxsize=None)
def _build(n_total: int, vocab: int, d: int, chunk: int):
    per_w = n_total // NUM_WORKERS
    n_chunks = per_w // chunk
    assert n_chunks % 2 == 0 and n_chunks >= 4
    n_groups = n_chunks // 2
    slices_per_row = d // LANES

    mesh = plsc.VectorSubcoreMesh(core_axis_name="c", subcore_axis_name="s")

    @functools.partial(
        pl.kernel,
        out_type=jax.ShapeDtypeStruct((n_total, d), jnp.float32),
        mesh=mesh,
        scratch_types=[
            pltpu.VMEM((per_w,), jnp.int32),
            pltpu.VMEM((2, chunk, d), jnp.float32),
            pltpu.VMEM((2, chunk, d), jnp.float32),
            pltpu.SemaphoreType.DMA,
            pltpu.SemaphoreType.DMA,
            pltpu.SemaphoreType.DMA,
            pltpu.SemaphoreType.DMA,
        ],
        compiler_params=pltpu.CompilerParams(use_tc_tiling_on_sc=False),
    )
    def emb_kernel(x_hbm, table_hbm, out_hbm, idx_v, gbuf, sbuf,
                   gsem0, gsem1, osem0, osem1):
        gsems = (gsem0, gsem1)
        osems = (osem0, osem1)
        wid = lax.axis_index("s") * NUM_CORES + lax.axis_index("c")
        base = wid * per_w
        pltpu.sync_copy(x_hbm.at[pl.ds(base, per_w)], idx_v)

        def start_gather(c, b, sem):
            pltpu.async_copy(
                table_hbm.at[idx_v.at[pl.ds(c * chunk, chunk)]],
                gbuf.at[b], sem)

        start_gather(0, 0, gsems[0])
        start_gather(1, 1, gsems[1])

        @pl.loop(0, n_groups)
        def group(g):
            for b in range(2):
                c = g * 2 + b
                pltpu.make_async_copy(
                    table_hbm.at[pl.ds(0, chunk)], gbuf.at[b],
                    gsems[b]).wait()
                @pl.when(g > 0)
                def _():
                    pltpu.make_async_copy(
                        sbuf.at[b], out_hbm.at[pl.ds(0, chunk)],
                        osems[b]).wait()

                @plsc.parallel_loop(0, chunk, unroll=4)
                def scale(r):
                    for j in range(slices_per_row):
                        sl = pl.ds(j * LANES, LANES)
                        sbuf[b, r, sl] = gbuf[b, r, sl] * SCALE

                @pl.when(g < n_groups - 1)
                def _():
                    start_gather(c + 2, b, gsems[b])

                pltpu.async_copy(
                    sbuf.at[b], out_hbm.at[pl.ds(base + c * chunk, chunk)],
                    osems[b])

        for b in range(2):
            pltpu.make_async_copy(
                sbuf.at[b], out_hbm.at[pl.ds(0, chunk)], osems[b]).wait()

    return emb_kernel


def _pick_chunk(per_w: int, target: int) -> int:
    best = None
    for c in range(8, per_w + 1, 8):
        if per_w % c or (per_w // c) % 2 or per_w // c < 4:
            continue
        if best is None or abs(c - target) < abs(best - target):
            best = c
    return best if best is not None else per_w


def kernel(x, table):
    orig_shape = x.shape
    xf = x.reshape(-1).astype(jnp.int32)
    n_total = xf.shape[0]
    vocab, d = table.shape
    assert n_total % NUM_WORKERS == 0
    per_w = n_total // NUM_WORKERS
    chunk = _pick_chunk(per_w, 320)
    out = _build(n_total, vocab, d, chunk)(xf, table)
    return out.reshape(orig_shape + (d,))


# speedup vs baseline: 1.0611x; 1.0190x over previous
import functools
import math

import jax
import jax.numpy as jnp
from jax import lax
from jax.experimental import pallas as pl
from jax.experimental.pallas import tpu as pltpu
from jax.experimental.pallas import tpu_sc as plsc

D_MODEL = 64
SCALE = math.sqrt(D_MODEL)
LANES = 16
NUM_CORES = 2
NUM_SUBCORES = 16
NUM_WORKERS = NUM_CORES * NUM_SUBCORES


@functools.lru_cache(maxsize=None)
def _build(n_total: int, vocab: int, d: int, chunk: int):
    per_w = n_total // NUM_WORKERS
    n_chunks = per_w // chunk
    assert n_chunks % 2 == 0 and n_chunks >= 4
    n_groups = n_chunks // 2
    slices_per_row = d // LANES

    mesh = plsc.VectorSubcoreMesh(core_axis_name="c", subcore_axis_name="s")

    @functools.partial(
        pl.kernel,
        out_type=jax.ShapeDtypeStruct((n_total, d), jnp.float32),
        mesh=mesh,
        scratch_types=[
            pltpu.VMEM((per_w,), jnp.int32),
            pltpu.VMEM((2, chunk, d), jnp.float32),
            pltpu.VMEM((2, chunk, d), jnp.float32),
            pltpu.SemaphoreType.DMA,
            pltpu.SemaphoreType.DMA,
            pltpu.SemaphoreType.DMA,
            pltpu.SemaphoreType.DMA,
        ],
        compiler_params=pltpu.CompilerParams(use_tc_tiling_on_sc=False),
    )
    def emb_kernel(x_hbm, table_hbm, out_hbm, idx_v, gbuf, sbuf,
                   gsem0, gsem1, osem0, osem1):
        gsems = (gsem0, gsem1)
        osems = (osem0, osem1)
        wid = lax.axis_index("s") * NUM_CORES + lax.axis_index("c")
        base = wid * per_w
        pltpu.sync_copy(x_hbm.at[pl.ds(base, per_w)], idx_v)

        def start_gather(c, b, sem):
            pltpu.async_copy(
                table_hbm.at[idx_v.at[pl.ds(c * chunk, chunk)]],
                gbuf.at[b], sem)

        start_gather(0, 0, gsems[0])
        start_gather(1, 1, gsems[1])

        @pl.loop(0, n_groups)
        def group(g):
            for b in range(2):
                c = g * 2 + b
                pltpu.make_async_copy(
                    table_hbm.at[pl.ds(0, chunk)], gbuf.at[b],
                    gsems[b]).wait()
                @pl.when(g > 0)
                def _():
                    pltpu.make_async_copy(
                        sbuf.at[b], out_hbm.at[pl.ds(0, chunk)],
                        osems[b]).wait()

                @plsc.parallel_loop(0, chunk, unroll=4)
                def scale(r):
                    for j in range(slices_per_row):
                        sl = pl.ds(j * LANES, LANES)
                        sbuf[b, r, sl] = gbuf[b, r, sl] * SCALE

                @pl.when(g < n_groups - 1)
                def _():
                    start_gather(c + 2, b, gsems[b])

                pltpu.async_copy(
                    sbuf.at[b], out_hbm.at[pl.ds(base + c * chunk, chunk)],
                    osems[b])

        for b in range(2):
            pltpu.make_async_copy(
                sbuf.at[b], out_hbm.at[pl.ds(0, chunk)], osems[b]).wait()

    return emb_kernel


def _pick_chunk(per_w: int, target: int) -> int:
    best = None
    for c in range(8, per_w + 1, 8):
        if per_w % c or (per_w // c) % 2 or per_w // c < 4:
            continue
        if best is None or abs(c - target) < abs(best - target):
            best = c
    return best if best is not None else per_w


def kernel(x, table):
    b_dim, s_dim = x.shape
    xf = x.T.reshape(-1).astype(jnp.int32)
    n_total = xf.shape[0]
    vocab, d = table.shape
    assert n_total % NUM_WORKERS == 0
    per_w = n_total // NUM_WORKERS
    chunk = _pick_chunk(per_w, 320)
    out = _build(n_total, vocab, d, chunk)(xf, table)
    return out.reshape(s_dim, b_dim, d).transpose(1, 0, 2)


# speedup vs baseline: 1.1150x; 1.0508x over previous
import functools
import math

import jax
import jax.numpy as jnp
from jax import lax
from jax.experimental import pallas as pl
from jax.experimental.pallas import tpu as pltpu
from jax.experimental.pallas import tpu_sc as plsc

D_MODEL = 64
SCALE = math.sqrt(D_MODEL)
LANES = 16
NUM_CORES = 2
NUM_SUBCORES = 16
NUM_WORKERS = NUM_CORES * NUM_SUBCORES


@functools.lru_cache(maxsize=None)
def _build(n_total: int, vocab2: int, d: int, chunk: int):
    per_w = n_total // NUM_WORKERS
    n_chunks = per_w // chunk
    assert n_chunks % 2 == 0 and n_chunks >= 4
    n_groups = n_chunks // 2
    slices_per_row = d // LANES

    mesh = plsc.VectorSubcoreMesh(core_axis_name="c", subcore_axis_name="s")

    @functools.partial(
        pl.kernel,
        out_type=jax.ShapeDtypeStruct((n_total, d), jnp.float32),
        mesh=mesh,
        scratch_types=[
            pltpu.VMEM((per_w + LANES,), jnp.int32),
            pltpu.VMEM((2, chunk), jnp.int32),
            pltpu.VMEM((2, chunk, 2 * d), jnp.float32),
            pltpu.VMEM((2, chunk, d), jnp.float32),
            pltpu.SemaphoreType.DMA,
            pltpu.SemaphoreType.DMA,
            pltpu.SemaphoreType.DMA,
            pltpu.SemaphoreType.DMA,
        ],
        compiler_params=pltpu.CompilerParams(use_tc_tiling_on_sc=True),
    )
    def emb_kernel(x_hbm, table_hbm, out_hbm, idx_v, bidx_v,
                   gbuf, sbuf, gsem0, gsem1, osem0, osem1):
        gsems = (gsem0, gsem1)
        osems = (osem0, osem1)
        wid = lax.axis_index("s") * NUM_CORES + lax.axis_index("c")
        base = wid * per_w
        pltpu.sync_copy(x_hbm.at[pl.ds(base, per_w)],
                        idx_v.at[pl.ds(0, per_w)])

        def fill_bidx_and_gather(c, b):
            cb = c * chunk

            @plsc.parallel_loop(0, chunk // LANES, unroll=4)
            def shift(k):
                sl = pl.ds(k * LANES, LANES)
                bidx_v[b, sl] = lax.shift_right_logical(
                    idx_v[pl.ds(cb + k * LANES, LANES)], 1)

            pltpu.async_copy(table_hbm.at[bidx_v.at[b]], gbuf.at[b],
                             gsems[b])

        fill_bidx_and_gather(0, 0)
        fill_bidx_and_gather(1, 1)

        @pl.loop(0, n_groups)
        def group(g):
            for b in range(2):
                c = g * 2 + b
                cb = c * chunk
                pltpu.make_async_copy(
                    table_hbm.at[pl.ds(0, chunk)], gbuf.at[b],
                    gsems[b]).wait()
                @pl.when(g > 0)
                def _():
                    pltpu.make_async_copy(
                        sbuf.at[b], out_hbm.at[pl.ds(0, chunk)],
                        osems[b]).wait()

                @plsc.parallel_loop(0, chunk, unroll=2)
                def scale(r):
                    half = (idx_v[pl.ds(cb + r, LANES)][0] & 1) * d
                    for j in range(slices_per_row):
                        sbuf[b, r, pl.ds(j * LANES, LANES)] = (
                            gbuf[b, r, pl.ds(half + j * LANES, LANES)]
                            * SCALE)

                @pl.when(g < n_groups - 1)
                def _():
                    fill_bidx_and_gather(c + 2, b)

                pltpu.async_copy(
                    sbuf.at[b], out_hbm.at[pl.ds(base + cb, chunk)],
                    osems[b])

        for b in range(2):
            pltpu.make_async_copy(
                sbuf.at[b], out_hbm.at[pl.ds(0, chunk)], osems[b]).wait()

    return emb_kernel


def _pick_chunk(per_w: int, target: int) -> int:
    best = None
    for c in range(LANES, min(per_w, 128) + 1, LANES):
        if per_w % c or (per_w // c) % 2 or per_w // c < 4:
            continue
        if best is None or abs(c - target) < abs(best - target):
            best = c
    return best if best is not None else per_w


def kernel(x, table):
    b_dim, s_dim = x.shape
    xf = x.T.reshape(-1).astype(jnp.int32)
    n_total = xf.shape[0]
    vocab, d = table.shape
    assert vocab % 2 == 0 and n_total % NUM_WORKERS == 0
    t2 = table.reshape(vocab // 2, 2 * d)
    per_w = n_total // NUM_WORKERS
    chunk = _pick_chunk(per_w, 128)
    out = _build(n_total, vocab // 2, d, chunk)(xf, t2)
    return out.reshape(s_dim, b_dim, d).transpose(1, 0, 2)


# speedup vs baseline: 2.3472x; 2.1051x over previous
import functools
import math

import jax
import jax.numpy as jnp
from jax import lax
from jax.experimental import pallas as pl
from jax.experimental.pallas import tpu as pltpu
from jax.experimental.pallas import tpu_sc as plsc

D_MODEL = 64
SCALE = math.sqrt(D_MODEL)
LANES = 16
NUM_CORES = 2
NUM_SUBCORES = 16
NUM_WORKERS = NUM_CORES * NUM_SUBCORES


@functools.lru_cache(maxsize=None)
def _build(n_total: int, vocab8: int, d: int, chunk: int):
    per_w = n_total // NUM_WORKERS
    n_chunks = per_w // chunk
    assert n_chunks % 2 == 0 and n_chunks >= 4
    n_groups = n_chunks // 2
    slices_per_row = d // LANES

    mesh = plsc.VectorSubcoreMesh(core_axis_name="c", subcore_axis_name="s")

    @functools.partial(
        pl.kernel,
        out_type=jax.ShapeDtypeStruct((n_total, d), jnp.float32),
        mesh=mesh,
        scratch_types=[
            pltpu.VMEM((per_w + LANES,), jnp.int32),
            pltpu.VMEM((2, chunk, d), jnp.float32),
            pltpu.VMEM((2, chunk, d), jnp.float32),
            pltpu.SemaphoreType.DMA,
            pltpu.SemaphoreType.DMA,
            pltpu.SemaphoreType.DMA,
            pltpu.SemaphoreType.DMA,
        ],
        compiler_params=pltpu.CompilerParams(use_tc_tiling_on_sc=True),
    )
    def emb_kernel(x_hbm, table_hbm, out_hbm, idx_v, gbuf, sbuf,
                   gsem0, gsem1, osem0, osem1):
        gsems = (gsem0, gsem1)
        osems = (osem0, osem1)
        wid = lax.axis_index("s") * NUM_CORES + lax.axis_index("c")
        base = wid * per_w
        pltpu.sync_copy(x_hbm.at[pl.ds(base, per_w)],
                        idx_v.at[pl.ds(0, per_w)])

        def fire_gather(c, b):
            cb = c * chunk

            @pl.loop(0, chunk // LANES)
            def issue(g):
                vec = idx_v[pl.ds(cb + g * LANES, LANES)]
                for l in range(LANES):
                    iv = vec[l]
                    pltpu.async_copy(
                        table_hbm.at[iv >> 3, iv & 7, :],
                        gbuf.at[b, g * LANES + l], gsems[b])

        fire_gather(0, 0)
        fire_gather(1, 1)

        @pl.loop(0, n_groups)
        def group(g):
            for b in range(2):
                c = g * 2 + b
                cb = c * chunk
                pltpu.make_async_copy(
                    out_hbm.at[pl.ds(0, chunk)], gbuf.at[b],
                    gsems[b]).wait()
                @pl.when(g > 0)
                def _():
                    pltpu.make_async_copy(
                        sbuf.at[b], out_hbm.at[pl.ds(0, chunk)],
                        osems[b]).wait()

                @plsc.parallel_loop(0, chunk, unroll=2)
                def scale(r):
                    for j in range(slices_per_row):
                        sl = pl.ds(j * LANES, LANES)
                        sbuf[b, r, sl] = gbuf[b, r, sl] * SCALE

                @pl.when(g < n_groups - 1)
                def _():
                    fire_gather(c + 2, b)

                pltpu.async_copy(
                    sbuf.at[b], out_hbm.at[pl.ds(base + cb, chunk)],
                    osems[b])

        for b in range(2):
            pltpu.make_async_copy(
                sbuf.at[b], out_hbm.at[pl.ds(0, chunk)], osems[b]).wait()

    return emb_kernel


def _pick_chunk(per_w: int, target: int) -> int:
    best = None
    for c in range(LANES, per_w + 1, LANES):
        if per_w % c or (per_w // c) % 2 or per_w // c < 4:
            continue
        if best is None or abs(c - target) < abs(best - target):
            best = c
    return best if best is not None else per_w


def kernel(x, table):
    b_dim, s_dim = x.shape
    xf = x.T.reshape(-1).astype(jnp.int32)
    n_total = xf.shape[0]
    vocab, d = table.shape
    assert vocab % 8 == 0 and n_total % NUM_WORKERS == 0
    t3 = table.reshape(vocab // 8, 8, d)
    per_w = n_total // NUM_WORKERS
    chunk = _pick_chunk(per_w, 128)
    out = _build(n_total, vocab // 8, d, chunk)(xf, t3)
    return out.reshape(s_dim, b_dim, d).transpose(1, 0, 2)


# speedup vs baseline: 2.3501x; 1.0012x over previous
import functools
import math

import jax
import jax.numpy as jnp
from jax import lax
from jax.experimental import pallas as pl
from jax.experimental.pallas import tpu as pltpu
from jax.experimental.pallas import tpu_sc as plsc

D_MODEL = 64
SCALE = math.sqrt(D_MODEL)
LANES = 16
NUM_CORES = 2
NUM_SUBCORES = 16
NUM_WORKERS = NUM_CORES * NUM_SUBCORES


@functools.lru_cache(maxsize=None)
def _build(n_total: int, vocab8: int, d: int, chunk: int):
    per_w = n_total // NUM_WORKERS
    n_chunks = per_w // chunk
    assert n_chunks % 2 == 0 and n_chunks >= 4
    n_groups = n_chunks // 2
    slices_per_row = d // LANES

    mesh = plsc.VectorSubcoreMesh(core_axis_name="c", subcore_axis_name="s")

    @functools.partial(
        pl.kernel,
        out_type=jax.ShapeDtypeStruct((n_total, d), jnp.float32),
        mesh=mesh,
        scratch_types=[
            pltpu.VMEM((per_w + LANES,), jnp.int32),
            pltpu.VMEM((2, chunk, d), jnp.float32),
            pltpu.VMEM((2, chunk, d), jnp.float32),
            pltpu.SemaphoreType.DMA,
            pltpu.SemaphoreType.DMA,
            pltpu.SemaphoreType.DMA,
            pltpu.SemaphoreType.DMA,
        ],
        compiler_params=pltpu.CompilerParams(use_tc_tiling_on_sc=True),
    )
    def emb_kernel(x_hbm, table_hbm, out_hbm, idx_v, gbuf, sbuf,
                   gsem0, gsem1, osem0, osem1):
        gsems = (gsem0, gsem1)
        osems = (osem0, osem1)
        wid = lax.axis_index("s") * NUM_CORES + lax.axis_index("c")
        base = wid * per_w
        pltpu.sync_copy(x_hbm.at[pl.ds(base, per_w)],
                        idx_v.at[pl.ds(0, per_w)])

        def fire_gather(c, b):
            cb = c * chunk

            @pl.loop(0, chunk // LANES)
            def issue(g):
                vec = idx_v[pl.ds(cb + g * LANES, LANES)]
                for l in range(LANES):
                    iv = vec[l]
                    pltpu.async_copy(
                        table_hbm.at[iv >> 3, iv & 7, :],
                        gbuf.at[b, g * LANES + l], gsems[b])

        fire_gather(0, 0)
        fire_gather(1, 1)

        @pl.loop(0, n_groups)
        def group(g):
            for b in range(2):
                c = g * 2 + b
                cb = c * chunk
                pltpu.make_async_copy(
                    out_hbm.at[pl.ds(0, chunk)], gbuf.at[b],
                    gsems[b]).wait()
                @pl.when(g > 0)
                def _():
                    pltpu.make_async_copy(
                        sbuf.at[b], out_hbm.at[pl.ds(0, chunk)],
                        osems[b]).wait()

                @plsc.parallel_loop(0, chunk, unroll=4)
                def scale(r):
                    for j in range(slices_per_row):
                        sl = pl.ds(j * LANES, LANES)
                        sbuf[b, r, sl] = gbuf[b, r, sl] * SCALE

                @pl.when(g < n_groups - 1)
                def _():
                    fire_gather(c + 2, b)

                pltpu.async_copy(
                    sbuf.at[b], out_hbm.at[pl.ds(base + cb, chunk)],
                    osems[b])

        for b in range(2):
            pltpu.make_async_copy(
                sbuf.at[b], out_hbm.at[pl.ds(0, chunk)], osems[b]).wait()

    return emb_kernel


def _pick_chunk(per_w: int, target: int) -> int:
    best = None
    for c in range(LANES, per_w + 1, LANES):
        if per_w % c or (per_w // c) % 2 or per_w // c < 4:
            continue
        if best is None or abs(c - target) < abs(best - target):
            best = c
    return best if best is not None else per_w


def kernel(x, table):
    b_dim, s_dim = x.shape
    xf = x.T.reshape(-1).astype(jnp.int32)
    n_total = xf.shape[0]
    vocab, d = table.shape
    assert vocab % 8 == 0 and n_total % NUM_WORKERS == 0
    t3 = table.reshape(vocab // 8, 8, d)
    per_w = n_total // NUM_WORKERS
    chunk = _pick_chunk(per_w, 160)
    out = _build(n_total, vocab // 8, d, chunk)(xf, t3)
    return out.reshape(s_dim, b_dim, d).transpose(1, 0, 2)
